# baseline (speedup 1.0000x reference)
import functools

import jax
import jax.numpy as jnp
from jax import lax
from jax.experimental import pallas as pl
from jax.experimental.pallas import tpu as pltpu
from jax.experimental.pallas import tpu_sc as plsc

N = 10000
E = 320000
C = 128
NB = 3

NC = 2
NS = 16
K = 112
STEPS0 = 120
STEPS1 = 59
JSTEPS = STEPS0 + 2
NPAIR0 = STEPS0 // 2
TOT0 = NS * STEPS0 * K
TOT1 = NS * STEPS1 * K
ROWS_PER_TILE = 632
NP = NS * ROWS_PER_TILE
SHIFT = 14
MASK = (1 << SHIFT) - 1


def _edge_agg_body(y_hbm, pk_hbm, zeros_hbm, out_hbm,
                   pk_v, su0, su1, du0, du1, rows0, rows1, acc,
                   g0, g1, s0, s1):
    c = lax.axis_index("c")
    s = lax.axis_index("s")
    r0 = s * ROWS_PER_TILE

    pltpu.sync_copy(pk_hbm.at[c, s], pk_v)
    pltpu.sync_copy(zeros_hbm.at[pl.ds(r0, ROWS_PER_TILE)],
                    acc.at[pl.ds(r0, ROWS_PER_TILE)])
    plsc.subcore_barrier()

    def unpack(j, su, du):
        for l in range(K // 16):
            v = pk_v[j, pl.ds(l * 16, 16)]
            su[0, pl.ds(l * 16, 16)] = lax.shift_right_logical(v, SHIFT)
            du[0, pl.ds(l * 16, 16)] = lax.bitwise_and(v, MASK)

    def wait(sem):
        pltpu.make_async_copy(y_hbm.at[pl.ds(0, K)], rows0, sem).wait()

    @pl.when(c == 0)
    def _pipelined():
        unpack(0, su0, du0)
        pltpu.async_copy(y_hbm.at[su0.at[0]], rows0, g0)
        for l in range(K // 16):
            du1[0, pl.ds(l * 16, 16)] = jnp.full((16,), N, jnp.int32)
        pltpu.async_copy(rows1, acc.at[du1.at[0]], s1, add=True)

        def pair(jj, carry):
            j0 = 2 * jj
            wait(g0)
            wait(s1)
            unpack(j0 + 1, su1, du1)
            pltpu.async_copy(y_hbm.at[su1.at[0]], rows1, g1)
            pltpu.async_copy(rows0, acc.at[du0.at[0]], s0, add=True)
            wait(g1)
            wait(s0)
            unpack(j0 + 2, su0, du0)
            pltpu.async_copy(y_hbm.at[su0.at[0]], rows0, g0)
            pltpu.async_copy(rows1, acc.at[du1.at[0]], s1, add=True)
            return carry

        lax.fori_loop(0, NPAIR0, pair, 0)
        wait(g0)
        wait(s1)

    @pl.when(c != 0)
    def _serial():
        def step(j, carry):
            unpack(j, su0, du0)
            pltpu.async_copy(y_hbm.at[su0.at[0]], rows0, g0).wait()
            pltpu.sync_copy(rows0, acc.at[du0.at[0]], add=True)
            return carry

        lax.fori_loop(0, STEPS1, step, 0)

    plsc.subcore_barrier()
    pltpu.sync_copy(acc.at[pl.ds(r0, ROWS_PER_TILE)],
                    out_hbm.at[c, pl.ds(r0, ROWS_PER_TILE)])


_edge_agg = pl.kernel(
    _edge_agg_body,
    out_type=jax.ShapeDtypeStruct((NC, NP, C), jnp.float32),
    mesh=plsc.VectorSubcoreMesh(core_axis_name="c", subcore_axis_name="s"),
    scratch_types=[
        pltpu.VMEM((JSTEPS, K), jnp.int32),
        pltpu.VMEM((1, K), jnp.int32),
        pltpu.VMEM((1, K), jnp.int32),
        pltpu.VMEM((1, K), jnp.int32),
        pltpu.VMEM((1, K), jnp.int32),
        pltpu.VMEM((K, C), jnp.float32),
        pltpu.VMEM((K, C), jnp.float32),
        pltpu.VMEM_SHARED((NP, C), jnp.float32),
        pltpu.SemaphoreType.DMA,
        pltpu.SemaphoreType.DMA,
        pltpu.SemaphoreType.DMA,
        pltpu.SemaphoreType.DMA,
    ],
)


def _mm2_body(h_ref, wn_ref, ws_ref, b_ref, y_ref, base_ref):
    h = h_ref[...]
    y_ref[...] = jnp.dot(h, wn_ref[...], preferred_element_type=jnp.float32)
    base_ref[...] = (jnp.dot(h, ws_ref[...], preferred_element_type=jnp.float32)
                     + b_ref[...])


_mm2 = pl.pallas_call(
    _mm2_body,
    out_shape=(jax.ShapeDtypeStruct((N, C), jnp.float32),
               jax.ShapeDtypeStruct((N, C), jnp.float32)),
)


def _bn_finish(parts_ref, base_ref, g_ref, be_ref, idn_ref, *, residual):
    t = parts_ref[0, :N, :] + parts_ref[1, :N, :] + base_ref[...]
    mu = jnp.mean(t, axis=0, keepdims=True)
    d = t - mu
    var = jnp.mean(d * d, axis=0, keepdims=True)
    out = d * lax.rsqrt(var + 1e-3) * g_ref[...] + be_ref[...]
    if residual:
        out = out + idn_ref[...]
    return jnp.maximum(out, 0.0)


def _bn_body(parts_ref, base_ref, g_ref, be_ref, idn_ref, o_ref, *, residual):
    o_ref[...] = _bn_finish(parts_ref, base_ref, g_ref, be_ref, idn_ref,
                            residual=residual)


_bn_res = pl.pallas_call(
    functools.partial(_bn_body, residual=True),
    out_shape=jax.ShapeDtypeStruct((N, C), jnp.float32),
)


def _bnmm_body(parts_ref, base_ref, g_ref, be_ref, idn_ref,
               wn_ref, ws_ref, b_ref, h_ref, y_ref, nbase_ref, *, residual):
    h = _bn_finish(parts_ref, base_ref, g_ref, be_ref, idn_ref,
                   residual=residual)
    h_ref[...] = h
    y_ref[...] = jnp.dot(h, wn_ref[...], preferred_element_type=jnp.float32)
    nbase_ref[...] = (jnp.dot(h, ws_ref[...],
                              preferred_element_type=jnp.float32) + b_ref[...])


def _make_bnmm(residual):
    return pl.pallas_call(
        functools.partial(_bnmm_body, residual=residual),
        out_shape=(jax.ShapeDtypeStruct((N, C), jnp.float32),
                   jax.ShapeDtypeStruct((N, C), jnp.float32),
                   jax.ShapeDtypeStruct((N, C), jnp.float32)),
    )


_bnmm_plain = _make_bnmm(False)
_bnmm_res = _make_bnmm(True)


def kernel(x, edge_index, Wn, Ws, b, gamma, beta):
    src = edge_index[0].astype(jnp.int32)
    dst = edge_index[1].astype(jnp.int32)
    pad = TOT0 + TOT1 - E
    src = jnp.concatenate([src, jnp.zeros((pad,), jnp.int32)])
    dst = jnp.concatenate([dst, jnp.full((pad,), N, jnp.int32)])

    pk = (src << SHIFT) | dst
    junkv = N
    pk0 = pk[:TOT0].reshape(1, NS, STEPS0, K)
    pk0 = jnp.pad(pk0, ((0, 0), (0, 0), (0, JSTEPS - STEPS0), (0, 0)),
                  constant_values=junkv)
    pk1 = pk[TOT0:].reshape(1, NS, STEPS1, K)
    pk1 = jnp.pad(pk1, ((0, 0), (0, 0), (0, JSTEPS - STEPS1), (0, 0)),
                  constant_values=junkv)
    pk_g = jnp.concatenate([pk0, pk1], axis=0)
    zeros = jnp.zeros((NP, C), jnp.float32)

    idn = x
    y, base = _mm2(x, Wn[0, 0], Ws[0, 0], b[0, 0][None])
    for t in range(2 * NB):
        i, j = divmod(t, 2)
        parts = _edge_agg(y, pk_g, zeros)
        if t == 2 * NB - 1:
            return _bn_res(parts, base, gamma[i, j][None], beta[i, j][None],
                           idn)
        ni, nj = divmod(t + 1, 2)
        fused = _bnmm_res if j == 1 else _bnmm_plain
        h, y, base = fused(parts, base, gamma[i, j][None], beta[i, j][None],
                           idn, Wn[ni, nj], Ws[ni, nj], b[ni, nj][None])
        if j == 1:
            idn = h


# speedup vs baseline: 1.0812x; 1.0812x over previous
import functools

import jax
import jax.numpy as jnp
from jax import lax
from jax.experimental import pallas as pl
from jax.experimental.pallas import tpu as pltpu
from jax.experimental.pallas import tpu_sc as plsc

N = 10000
E = 320000
C = 128
NB = 3

NC = 2
NS = 16
K = 128
STEPS0 = 98
STEPS1 = 59
TOT0 = NS * STEPS0 * K
TOT1 = NS * STEPS1 * K
ROWS_PER_TILE = 632
NP = NS * ROWS_PER_TILE


def _edge_agg_body(y_hbm, src_hbm, dst_hbm, zeros_hbm, out_hbm,
                   src_v, dst_v, rows_v, acc, g0):
    c = lax.axis_index("c")
    s = lax.axis_index("s")
    r0 = s * ROWS_PER_TILE

    pltpu.sync_copy(src_hbm.at[c, s], src_v)
    pltpu.sync_copy(dst_hbm.at[c, s], dst_v)
    pltpu.sync_copy(zeros_hbm.at[pl.ds(r0, ROWS_PER_TILE)],
                    acc.at[pl.ds(r0, ROWS_PER_TILE)])
    plsc.subcore_barrier()

    nsteps = jnp.where(c == 0, STEPS0, STEPS1)

    def step(j, carry):
        pltpu.async_copy(y_hbm.at[src_v.at[j]], rows_v, g0).wait()
        pltpu.sync_copy(rows_v, acc.at[dst_v.at[j]], add=True)
        return carry

    lax.fori_loop(0, nsteps, step, 0)
    plsc.subcore_barrier()
    pltpu.sync_copy(acc.at[pl.ds(r0, ROWS_PER_TILE)],
                    out_hbm.at[c, pl.ds(r0, ROWS_PER_TILE)])


_edge_agg = pl.kernel(
    _edge_agg_body,
    out_type=jax.ShapeDtypeStruct((NC, NP, C), jnp.float32),
    mesh=plsc.VectorSubcoreMesh(core_axis_name="c", subcore_axis_name="s"),
    scratch_types=[
        pltpu.VMEM((STEPS0, K), jnp.int32),
        pltpu.VMEM((STEPS0, K), jnp.int32),
        pltpu.VMEM((K, C), jnp.float32),
        pltpu.VMEM_SHARED((NP, C), jnp.float32),
        pltpu.SemaphoreType.DMA,
    ],
)


def _mm2_body(h_ref, wn_ref, ws_ref, b_ref, y_ref, base_ref):
    h = h_ref[...]
    y_ref[...] = jnp.dot(h, wn_ref[...], preferred_element_type=jnp.float32)
    base_ref[...] = (jnp.dot(h, ws_ref[...], preferred_element_type=jnp.float32)
                     + b_ref[...])


_mm2 = pl.pallas_call(
    _mm2_body,
    out_shape=(jax.ShapeDtypeStruct((N, C), jnp.float32),
               jax.ShapeDtypeStruct((N, C), jnp.float32)),
)


def _bn_finish(parts_ref, base_ref, g_ref, be_ref, idn_ref, *, residual):
    t = parts_ref[0, :N, :] + parts_ref[1, :N, :] + base_ref[...]
    mu = jnp.mean(t, axis=0, keepdims=True)
    d = t - mu
    var = jnp.mean(d * d, axis=0, keepdims=True)
    out = d * lax.rsqrt(var + 1e-3) * g_ref[...] + be_ref[...]
    if residual:
        out = out + idn_ref[...]
    return jnp.maximum(out, 0.0)


def _bn_body(parts_ref, base_ref, g_ref, be_ref, idn_ref, o_ref, *, residual):
    o_ref[...] = _bn_finish(parts_ref, base_ref, g_ref, be_ref, idn_ref,
                            residual=residual)


_bn_res = pl.pallas_call(
    functools.partial(_bn_body, residual=True),
    out_shape=jax.ShapeDtypeStruct((N, C), jnp.float32),
)


def _bnmm_body(parts_ref, base_ref, g_ref, be_ref, idn_ref,
               wn_ref, ws_ref, b_ref, h_ref, y_ref, nbase_ref, *, residual):
    h = _bn_finish(parts_ref, base_ref, g_ref, be_ref, idn_ref,
                   residual=residual)
    h_ref[...] = h
    y_ref[...] = jnp.dot(h, wn_ref[...], preferred_element_type=jnp.float32)
    nbase_ref[...] = (jnp.dot(h, ws_ref[...],
                              preferred_element_type=jnp.float32) + b_ref[...])


def _make_bnmm(residual):
    return pl.pallas_call(
        functools.partial(_bnmm_body, residual=residual),
        out_shape=(jax.ShapeDtypeStruct((N, C), jnp.float32),
                   jax.ShapeDtypeStruct((N, C), jnp.float32),
                   jax.ShapeDtypeStruct((N, C), jnp.float32)),
    )


_bnmm_plain = _make_bnmm(False)
_bnmm_res = _make_bnmm(True)


def kernel(x, edge_index, Wn, Ws, b, gamma, beta):
    src = edge_index[0].astype(jnp.int32)
    dst = edge_index[1].astype(jnp.int32)
    pad = TOT0 + TOT1 - E
    src = jnp.concatenate([src, jnp.zeros((pad,), jnp.int32)])
    dst = jnp.concatenate([dst, jnp.full((pad,), N, jnp.int32)])

    def _core_layout(a):
        a0 = a[:TOT0].reshape(1, NS, STEPS0, K)
        a1 = a[TOT0:].reshape(1, NS, STEPS1, K)
        a1 = jnp.pad(a1, ((0, 0), (0, 0), (0, STEPS0 - STEPS1), (0, 0)))
        return jnp.concatenate([a0, a1], axis=0)

    src_g = _core_layout(src)
    dst_g = _core_layout(dst)
    zeros = jnp.zeros((NP, C), jnp.float32)

    idn = x
    y, base = _mm2(x, Wn[0, 0], Ws[0, 0], b[0, 0][None])
    for t in range(2 * NB):
        i, j = divmod(t, 2)
        parts = _edge_agg(y, src_g, dst_g, zeros)
        if t == 2 * NB - 1:
            return _bn_res(parts, base, gamma[i, j][None], beta[i, j][None],
                           idn)
        ni, nj = divmod(t + 1, 2)
        fused = _bnmm_res if j == 1 else _bnmm_plain
        h, y, base = fused(parts, base, gamma[i, j][None], beta[i, j][None],
                           idn, Wn[ni, nj], Ws[ni, nj], b[ni, nj][None])
        if j == 1:
            idn = h
